# baseline (speedup 1.0000x reference)
import functools

import jax
import jax.numpy as jnp
from jax import lax
from jax.experimental import pallas as pl
from jax.experimental.pallas import tpu as pltpu
from jax.experimental.pallas import tpu_sc as plsc

B = 4096
L = 50
D = 32
PK = 128 // D
NC = 2
NS = 16
NW = NC * NS
BPW = B // NW
G = 16
NG = BPW // G
GL = G * L
CHUNKS = [(0, 128), (128, 128), (256, 128), (384, 128),
          (512, 128), (640, 128), (768, 32)]

_mesh = plsc.VectorSubcoreMesh(core_axis_name="c", subcore_axis_name="s")

_TBLK = 65536
_SBLK = _TBLK // PK
_SB_SH = _SBLK.bit_length() - 1
_TB_SH = _TBLK.bit_length() - 1


def _packed_row(v):
    return ((v >> _TB_SH) << _SB_SH) + (v & (_SBLK - 1))


def _packed_col(v):
    return ((v >> _SB_SH) & (PK - 1)) * D


def _sc_feature(ctx_flat, nxt_flat, tab4):
    @functools.partial(
        pl.kernel,
        out_type=jax.ShapeDtypeStruct((B,), jnp.float32),
        mesh=_mesh,
        compiler_params=pltpu.CompilerParams(
            use_tc_tiling_on_sc=True,
            needs_layout_passes=False,
        ),
        scratch_types=[
            pltpu.VMEM((GL,), jnp.int32),
            pltpu.VMEM((GL,), jnp.int32),
            pltpu.VMEM((GL,), jnp.int32),
            pltpu.VMEM((G,), jnp.int32),
            pltpu.VMEM((G,), jnp.int32),
            pltpu.VMEM((GL, 128), jnp.float32),
            pltpu.VMEM((G, 128), jnp.float32),
            pltpu.VMEM((BPW,), jnp.float32),
            pltpu.SemaphoreType.DMA,
        ],
    )
    def body(ctx_hbm, nxt_hbm, tab_hbm, out_hbm,
             cidx, crow4, ccol, nidx, nrow4, crows, nrows, outbuf, sem):
        wid = lax.axis_index("s") * NC + lax.axis_index("c")
        lane = lax.iota(jnp.int32, G)

        def group_body(g, carry):
            b0 = wid * BPW + g * G
            pltpu.sync_copy(ctx_hbm.at[pl.ds(pl.multiple_of(b0 * L, GL), GL)],
                            cidx)
            pltpu.sync_copy(nxt_hbm.at[pl.ds(pl.multiple_of(b0, G), G)], nidx)
            for j in range(GL // G):
                iv = cidx[pl.ds(j * G, G)]
                crow4[pl.ds(j * G, G)] = _packed_row(iv)
                ccol[pl.ds(j * G, G)] = _packed_col(iv)
            niv = nidx[...]
            nrow4[...] = _packed_row(niv)
            ncol = _packed_col(niv)

            copies = [pltpu.make_async_copy(tab_hbm.at[nrow4], nrows, sem)]
            for (s, n) in CHUNKS:
                copies.append(pltpu.make_async_copy(
                    tab_hbm.at[crow4.at[pl.ds(s, n)]],
                    crows.at[pl.ds(s, n)], sem))
            for c in copies:
                c.start()
            for c in copies:
                c.wait()

            def l_body(l, m):
                row_ids = lane * L + l
                cb = plsc.load_gather(ccol, [row_ids])
                acc = jnp.zeros((G,), jnp.float32)
                for d in range(D):
                    col = plsc.load_gather(crows, [row_ids, cb + d])
                    nv = plsc.load_gather(nrows, [lane, ncol + d])
                    acc = acc + col * nv
                return jnp.maximum(m, acc)

            m = lax.fori_loop(0, L, l_body,
                              jnp.full((G,), -jnp.inf, jnp.float32))
            outbuf[pl.ds(g * G, G)] = m
            return carry

        lax.fori_loop(0, NG, group_body, 0)
        pltpu.sync_copy(outbuf,
                        out_hbm.at[pl.ds(pl.multiple_of(wid * BPW, BPW), BPW)])

    return body(ctx_flat, nxt_flat, tab4)


def _tc_pack(tab_t):
    v = tab_t.shape[1]
    nb = (v + _TBLK - 1) // _TBLK

    def body(x_ref, o_ref):
        r = lax.broadcasted_iota(jnp.int32, (D, D), 0)
        c = lax.broadcasted_iota(jnp.int32, (D, D), 1)
        ident = (r == c).astype(jnp.float32)
        x = x_ref[...]
        parts = [
            lax.dot_general(x[:, a * _SBLK:(a + 1) * _SBLK], ident,
                            (((0,), (0,)), ((), ())),
                            preferred_element_type=jnp.float32)
            for a in range(PK)
        ]
        o_ref[...] = jnp.concatenate(parts, axis=1)

    return pl.pallas_call(
        body,
        grid=(nb,),
        in_specs=[pl.BlockSpec((D, _TBLK), lambda i: (0, i))],
        out_specs=pl.BlockSpec((_SBLK, D * PK), lambda i: (i, 0)),
        out_shape=jax.ShapeDtypeStruct((nb * _SBLK, D * PK), jnp.float32),
        compiler_params=pltpu.CompilerParams(
            vmem_limit_bytes=100 * 1024 * 1024),
    )(tab_t)


def kernel(track_context, artist_context, album_context, next_track,
           next_artist, next_album, track_table, artist_table, album_table):
    tctx = track_context.reshape(-1).astype(jnp.int32)
    actx = artist_context.reshape(-1).astype(jnp.int32)
    bctx = album_context.reshape(-1).astype(jnp.int32)
    tnxt = next_track.reshape(-1).astype(jnp.int32)
    anxt = next_artist.reshape(-1).astype(jnp.int32)
    bnxt = next_album.reshape(-1).astype(jnp.int32)
    atab = _tc_pack(artist_table.T)
    bt, _ = lax.optimization_barrier((album_table.T, atab))
    btab = _tc_pack(bt)
    tt, _ = lax.optimization_barrier((track_table.T, btab))
    ttab = _tc_pack(tt)
    aout = _sc_feature(actx, anxt, atab)
    bctx2, _ = lax.optimization_barrier((bctx, aout))
    bout = _sc_feature(bctx2, bnxt, btab)
    tctx2, _ = lax.optimization_barrier((tctx, bout))
    tout = _sc_feature(tctx2, tnxt, ttab)
    return (tout, aout, bout)


# speedup vs baseline: 1.0192x; 1.0192x over previous
import functools

import jax
import jax.numpy as jnp
from jax import lax
from jax.experimental import pallas as pl
from jax.experimental.pallas import tpu as pltpu
from jax.experimental.pallas import tpu_sc as plsc

B = 4096
L = 50
D = 32
PK = 128 // D
NC = 2
NS = 16
NW = NC * NS
BPW = B // NW
G = 16
NG = BPW // G
GL = G * L
CHUNKS = [(0, 128), (128, 128), (256, 128), (384, 128),
          (512, 128), (640, 128), (768, 32)]

_mesh = plsc.VectorSubcoreMesh(core_axis_name="c", subcore_axis_name="s")

_TBLK = 16384
_SBLK = _TBLK // PK
_SB_SH = _SBLK.bit_length() - 1
_TB_SH = _TBLK.bit_length() - 1


def _packed_row(v):
    return ((v >> _TB_SH) << _SB_SH) + (v & (_SBLK - 1))


def _packed_col(v):
    return ((v >> _SB_SH) & (PK - 1)) * D


def _sc_feature(ctx_flat, nxt_flat, tab4):
    @functools.partial(
        pl.kernel,
        out_type=jax.ShapeDtypeStruct((B,), jnp.float32),
        mesh=_mesh,
        compiler_params=pltpu.CompilerParams(
            use_tc_tiling_on_sc=True,
            needs_layout_passes=False,
        ),
        scratch_types=[
            pltpu.VMEM((GL,), jnp.int32),
            pltpu.VMEM((GL,), jnp.int32),
            pltpu.VMEM((GL,), jnp.int32),
            pltpu.VMEM((G,), jnp.int32),
            pltpu.VMEM((G,), jnp.int32),
            pltpu.VMEM((GL, 128), jnp.float32),
            pltpu.VMEM((G, 128), jnp.float32),
            pltpu.VMEM((BPW,), jnp.float32),
            pltpu.SemaphoreType.DMA,
        ],
    )
    def body(ctx_hbm, nxt_hbm, tab_hbm, out_hbm,
             cidx, crow4, ccol, nidx, nrow4, crows, nrows, outbuf, sem):
        wid = lax.axis_index("s") * NC + lax.axis_index("c")
        lane = lax.iota(jnp.int32, G)

        def group_body(g, carry):
            b0 = wid * BPW + g * G
            pltpu.sync_copy(ctx_hbm.at[pl.ds(pl.multiple_of(b0 * L, GL), GL)],
                            cidx)
            pltpu.sync_copy(nxt_hbm.at[pl.ds(pl.multiple_of(b0, G), G)], nidx)
            for j in range(GL // G):
                iv = cidx[pl.ds(j * G, G)]
                crow4[pl.ds(j * G, G)] = _packed_row(iv)
                ccol[pl.ds(j * G, G)] = _packed_col(iv)
            niv = nidx[...]
            nrow4[...] = _packed_row(niv)
            ncol = _packed_col(niv)

            copies = [pltpu.make_async_copy(tab_hbm.at[nrow4], nrows, sem)]
            for (s, n) in CHUNKS:
                copies.append(pltpu.make_async_copy(
                    tab_hbm.at[crow4.at[pl.ds(s, n)]],
                    crows.at[pl.ds(s, n)], sem))
            for c in copies:
                c.start()
            for c in copies:
                c.wait()

            def l_body(l, m):
                row_ids = lane * L + l
                cb = plsc.load_gather(ccol, [row_ids])
                acc = jnp.zeros((G,), jnp.float32)
                for d in range(D):
                    col = plsc.load_gather(crows, [row_ids, cb + d])
                    nv = plsc.load_gather(nrows, [lane, ncol + d])
                    acc = acc + col * nv
                return jnp.maximum(m, acc)

            m = lax.fori_loop(0, L, l_body,
                              jnp.full((G,), -jnp.inf, jnp.float32))
            outbuf[pl.ds(g * G, G)] = m
            return carry

        lax.fori_loop(0, NG, group_body, 0)
        pltpu.sync_copy(outbuf,
                        out_hbm.at[pl.ds(pl.multiple_of(wid * BPW, BPW), BPW)])

    return body(ctx_flat, nxt_flat, tab4)


def _tc_pack(tab_t):
    v = tab_t.shape[1]
    nb = (v + _TBLK - 1) // _TBLK

    def body(x_ref, o_ref):
        r = lax.broadcasted_iota(jnp.int32, (D, D), 0)
        c = lax.broadcasted_iota(jnp.int32, (D, D), 1)
        ident = (r == c).astype(jnp.float32)
        x = x_ref[...]
        parts = [
            lax.dot_general(x[:, a * _SBLK:(a + 1) * _SBLK], ident,
                            (((0,), (0,)), ((), ())),
                            preferred_element_type=jnp.float32)
            for a in range(PK)
        ]
        o_ref[...] = jnp.concatenate(parts, axis=1)

    return pl.pallas_call(
        body,
        grid=(nb,),
        in_specs=[pl.BlockSpec((D, _TBLK), lambda i: (0, i))],
        out_specs=pl.BlockSpec((_SBLK, D * PK), lambda i: (i, 0)),
        out_shape=jax.ShapeDtypeStruct((nb * _SBLK, D * PK), jnp.float32),
    )(tab_t)


def kernel(track_context, artist_context, album_context, next_track,
           next_artist, next_album, track_table, artist_table, album_table):
    tctx = track_context.reshape(-1).astype(jnp.int32)
    actx = artist_context.reshape(-1).astype(jnp.int32)
    bctx = album_context.reshape(-1).astype(jnp.int32)
    tnxt = next_track.reshape(-1).astype(jnp.int32)
    anxt = next_artist.reshape(-1).astype(jnp.int32)
    bnxt = next_album.reshape(-1).astype(jnp.int32)
    atab = _tc_pack(artist_table.T)
    bt, _ = lax.optimization_barrier((album_table.T, atab))
    btab = _tc_pack(bt)
    tt, _ = lax.optimization_barrier((track_table.T, btab))
    ttab = _tc_pack(tt)
    aout = _sc_feature(actx, anxt, atab)
    bctx2, _ = lax.optimization_barrier((bctx, aout))
    bout = _sc_feature(bctx2, bnxt, btab)
    tctx2, _ = lax.optimization_barrier((tctx, bout))
    tout = _sc_feature(tctx2, tnxt, ttab)
    return (tout, aout, bout)
